# Initial kernel scaffold; baseline (speedup 1.0000x reference)
import jax
import jax.numpy as jnp
from jax.experimental import pallas as pl


def kernel(x, idx_swap):
    raise NotImplementedError("write your pallas kernel here")


# speedup vs baseline: 1.0325x; 1.0325x over previous
import jax
import jax.numpy as jnp
from jax.experimental import pallas as pl
from jax.experimental.pallas import tpu as pltpu

EPS = 1e-05


def _spe_kernel(s_ref, xs_ref, xn_ref, out_ref):
    C = xn_ref.shape[1]
    xs = xs_ref[0]
    xn = xn_ref[0]

    sum_n = jnp.sum(xn, axis=0, keepdims=True)
    sumsq_n = jnp.sum(xn * xn, axis=0, keepdims=True)
    mean_n = sum_n * (1.0 / C)
    var_n = (sumsq_n - sum_n * mean_n) * (1.0 / (C - 1))

    sum_s = jnp.sum(xs, axis=0, keepdims=True)
    sumsq_s = jnp.sum(xs * xs, axis=0, keepdims=True)
    mean_s = sum_s * (1.0 / C)
    var_s = (sumsq_s - sum_s * mean_s) * (1.0 / (C - 1))

    ratio = jnp.sqrt((var_n + EPS) / (var_s + EPS))
    out_ref[0] = (xs - mean_s) * ratio + mean_n


def kernel(x, idx_swap):
    N, C, H, W = x.shape
    HW = H * W
    xv = x.reshape(N, C, HW)

    grid_spec = pltpu.PrefetchScalarGridSpec(
        num_scalar_prefetch=1,
        grid=(N,),
        in_specs=[
            pl.BlockSpec((1, C, HW), lambda n, s: (s[n], 0, 0)),
            pl.BlockSpec((1, C, HW), lambda n, s: (n, 0, 0)),
        ],
        out_specs=pl.BlockSpec((1, C, HW), lambda n, s: (n, 0, 0)),
    )

    out = pl.pallas_call(
        _spe_kernel,
        grid_spec=grid_spec,
        out_shape=jax.ShapeDtypeStruct((N, C, HW), jnp.float32),
    )(idx_swap, xv, xv)
    return out.reshape(N, C, H, W)
